# Initial kernel scaffold; baseline (speedup 1.0000x reference)
import functools

import jax
import jax.numpy as jnp
from jax import lax
from jax.experimental import pallas as pl
from jax.experimental.pallas import tpu as pltpu, tpu_sc as plsc

_N = 10000
_D = 128
_E = 320000
_EPS = 1e-5

_NP = 10240
_CH = 128
_NCHUNK = 79
_EPT = _CH * _NCHUNK
_EPAD = 32 * _EPT
_RPT = _NP // 16

_mesh = plsc.VectorSubcoreMesh(core_axis_name="c", subcore_axis_name="s")


@functools.partial(
    pl.kernel,
    mesh=_mesh,
    out_type=jax.ShapeDtypeStruct((2, _NP, 1), jnp.float32),
    scratch_types=[
        pltpu.VMEM((_CH,), jnp.int32),
        pltpu.VMEM((_CH,), jnp.float32),
        pltpu.VMEM((_RPT,), jnp.float32),
        pltpu.VMEM_SHARED((_NP,), jnp.float32),
    ],
)
def _sc_counts(dst_hbm, out_hbm, idx_v, ones_v, zero_v, acc_sh):
    cid = lax.axis_index("c")
    sid = lax.axis_index("s")
    wid = sid * 2 + cid
    for k in range(_CH // 16):
        ones_v[pl.ds(k * 16, 16)] = jnp.ones((16,), jnp.float32)

    def zbody(k, carry):
        zero_v[pl.ds(k * 16, 16)] = jnp.zeros((16,), jnp.float32)
        return carry

    lax.fori_loop(0, _RPT // 16, zbody, 0)
    pltpu.sync_copy(zero_v, acc_sh.at[pl.ds(sid * _RPT, _RPT)])
    plsc.subcore_barrier()

    def body(j, carry):
        base = wid * _EPT + j * _CH
        pltpu.sync_copy(dst_hbm.at[pl.ds(base, _CH)], idx_v)
        pltpu.sync_copy(ones_v, acc_sh.at[idx_v], add=True)
        return carry

    lax.fori_loop(0, _NCHUNK, body, 0)
    plsc.subcore_barrier()
    pltpu.sync_copy(acc_sh.at[pl.ds(sid * _RPT, _RPT)],
                    out_hbm.at[cid, pl.ds(sid * _RPT, _RPT), 0])


@functools.partial(
    pl.kernel,
    mesh=_mesh,
    out_type=jax.ShapeDtypeStruct((2, _NP, _D), jnp.float32),
    scratch_types=[
        pltpu.VMEM((_CH,), jnp.int32),
        pltpu.VMEM((_CH,), jnp.int32),
        pltpu.VMEM((_CH, _D), jnp.float32),
        pltpu.VMEM_SHARED((_NP, _D), jnp.float32),
        pltpu.SemaphoreType.DMA,
    ],
)
def _sc_spmm(xp_hbm, src_hbm, dst_hbm, out_hbm,
             src_v, dst_v, rows_v, acc_sh, sem):
    cid = lax.axis_index("c")
    sid = lax.axis_index("s")
    wid = sid * 2 + cid

    def zbody(r, carry):
        for k in range(_D // 16):
            rows_v[r, pl.ds(k * 16, 16)] = jnp.zeros((16,), jnp.float32)
        return carry

    lax.fori_loop(0, _CH, zbody, 0)
    for k in range(_RPT // _CH):
        pltpu.sync_copy(rows_v, acc_sh.at[pl.ds(sid * _RPT + k * _CH, _CH), :])
    plsc.subcore_barrier()

    def body(j, carry):
        base = wid * _EPT + j * _CH
        pltpu.sync_copy(src_hbm.at[pl.ds(base, _CH)], src_v)
        pltpu.sync_copy(dst_hbm.at[pl.ds(base, _CH)], dst_v)
        pltpu.async_copy(xp_hbm.at[src_v], rows_v, sem).wait()
        pltpu.sync_copy(rows_v, acc_sh.at[dst_v], add=True)
        return carry

    lax.fori_loop(0, _NCHUNK, body, 0)
    plsc.subcore_barrier()
    pltpu.sync_copy(acc_sh.at[pl.ds(sid * _RPT, _RPT), :],
                    out_hbm.at[cid, pl.ds(sid * _RPT, _RPT), :])


def _tc_prep_body(cnt_ref, x_ref, dinv_ref, xp_ref):
    deg = cnt_ref[0] + cnt_ref[1] + 1.0
    dinv = lax.rsqrt(deg)
    dinv_ref[...] = dinv
    xp_ref[...] = x_ref[...] * dinv


_tc_prep = pl.pallas_call(
    _tc_prep_body,
    out_shape=(
        jax.ShapeDtypeStruct((_NP, 1), jnp.float32),
        jax.ShapeDtypeStruct((_NP, _D), jnp.float32),
    ),
)


def _tc_layer_body(p_ref, xp_ref, dinv_ref, w_ref, b_ref, g_ref, bt_ref,
                   o_ref):
    dinv = dinv_ref[...]
    t = (p_ref[0] + p_ref[1] + xp_ref[...]) * dinv
    y = jnp.dot(t, w_ref[...], preferred_element_type=jnp.float32) + b_ref[...]
    yr = y[:_N]
    m = jnp.mean(yr, axis=0, keepdims=True)
    v = jnp.mean((yr - m) * (yr - m), axis=0, keepdims=True)
    h = (y - m) * lax.rsqrt(v + _EPS) * g_ref[...] + bt_ref[...]
    o_ref[...] = jnp.maximum(h, 0.0) * dinv


_tc_layer = pl.pallas_call(
    _tc_layer_body,
    out_shape=jax.ShapeDtypeStruct((_NP, _D), jnp.float32),
)


def _tc_final_body(p_ref, xp_ref, dinv_ref, w_ref, b_ref, x_ref, o_ref):
    t = (p_ref[0] + p_ref[1] + xp_ref[...]) * dinv_ref[...]
    y = jnp.dot(t, w_ref[...], preferred_element_type=jnp.float32)
    o_ref[...] = y + b_ref[...] + x_ref[...]


_tc_final = pl.pallas_call(
    _tc_final_body,
    out_shape=jax.ShapeDtypeStruct((_NP, _D), jnp.float32),
)


def kernel(x, edge_index, W1, b1, g1, bt1, W2, b2, g2, bt2, W3, b3):
    src = edge_index[0].astype(jnp.int32)
    dst = edge_index[1].astype(jnp.int32)
    pad = jnp.full((_EPAD - _E,), _N, jnp.int32)
    srcp = jnp.concatenate([src, pad])
    dstp = jnp.concatenate([dst, pad])
    xpd = jnp.pad(x, ((0, _NP - _N), (0, 0)))

    cnt = _sc_counts(dstp)
    dinv, xp1 = _tc_prep(cnt, xpd)
    p1 = _sc_spmm(xp1, srcp, dstp)
    xp2 = _tc_layer(p1, xp1, dinv, W1, b1.reshape(1, _D),
                    g1.reshape(1, _D), bt1.reshape(1, _D))
    p2 = _sc_spmm(xp2, srcp, dstp)
    xp3 = _tc_layer(p2, xp2, dinv, W2, b2.reshape(1, _D),
                    g2.reshape(1, _D), bt2.reshape(1, _D))
    p3 = _sc_spmm(xp3, srcp, dstp)
    out = _tc_final(p3, xp3, dinv, W3, b3.reshape(1, _D), xpd)
    return out[:_N]


# speedup vs baseline: 7.9468x; 7.9468x over previous
import functools

import jax
import jax.numpy as jnp
from jax import lax
from jax.experimental import pallas as pl
from jax.experimental.pallas import tpu as pltpu, tpu_sc as plsc

_N = 10000
_D = 128
_E = 320000
_EPS = 1e-5

_NP = 10240
_CH = 128
_NCHUNK = 79
_EPT = _CH * _NCHUNK
_EPAD = 32 * _EPT
_RPT = _NP // 16

_SC_CACHE = {}


def _sc_kernels():
    if "k" in _SC_CACHE:
        return _SC_CACHE["k"]
    mesh = plsc.VectorSubcoreMesh(core_axis_name="c", subcore_axis_name="s")

    @functools.partial(
        pl.kernel,
        mesh=mesh,
        out_type=jax.ShapeDtypeStruct((2, _NP), jnp.float32),
        scratch_types=[
            pltpu.VMEM((_CH,), jnp.int32),
            pltpu.VMEM((_CH,), jnp.float32),
            pltpu.VMEM((_RPT,), jnp.float32),
            pltpu.VMEM_SHARED((_NP,), jnp.float32),
        ],
    )
    def sc_counts(dst_hbm, out_hbm, idx_v, ones_v, zero_v, acc_sh):
        cid = lax.axis_index("c")
        sid = lax.axis_index("s")
        wid = sid * 2 + cid
        for k in range(_CH // 16):
            ones_v[pl.ds(k * 16, 16)] = jnp.ones((16,), jnp.float32)

        def zbody(k, carry):
            zero_v[pl.ds(k * 16, 16)] = jnp.zeros((16,), jnp.float32)
            return carry

        lax.fori_loop(0, _RPT // 16, zbody, 0)
        pltpu.sync_copy(zero_v, acc_sh.at[pl.ds(sid * _RPT, _RPT)])
        plsc.subcore_barrier()

        def body(j, carry):
            base = wid * _EPT + j * _CH
            pltpu.sync_copy(dst_hbm.at[pl.ds(base, _CH)], idx_v)
            pltpu.sync_copy(ones_v, acc_sh.at[idx_v], add=True)
            return carry

        lax.fori_loop(0, _NCHUNK, body, 0)
        plsc.subcore_barrier()
        pltpu.sync_copy(acc_sh.at[pl.ds(sid * _RPT, _RPT)],
                        out_hbm.at[cid, pl.ds(sid * _RPT, _RPT)])

    @functools.partial(
        pl.kernel,
        mesh=mesh,
        out_type=jax.ShapeDtypeStruct((2, _NP, _D), jnp.float32),
        scratch_types=[
            pltpu.VMEM((_CH,), jnp.int32),
            pltpu.VMEM((_CH,), jnp.int32),
            pltpu.VMEM((_CH, _D), jnp.float32),
            pltpu.VMEM_SHARED((_NP, _D), jnp.float32),
            pltpu.SemaphoreType.DMA,
        ],
    )
    def sc_spmm(xp_hbm, src_hbm, dst_hbm, out_hbm,
                src_v, dst_v, rows_v, acc_sh, sem):
        cid = lax.axis_index("c")
        sid = lax.axis_index("s")
        wid = sid * 2 + cid

        def zbody(r, carry):
            for k in range(_D // 16):
                rows_v[r, pl.ds(k * 16, 16)] = jnp.zeros((16,), jnp.float32)
            return carry

        lax.fori_loop(0, _CH, zbody, 0)
        for k in range(_RPT // _CH):
            pltpu.sync_copy(rows_v,
                            acc_sh.at[pl.ds(sid * _RPT + k * _CH, _CH), :])
        plsc.subcore_barrier()

        def body(j, carry):
            base = wid * _EPT + j * _CH
            pltpu.sync_copy(src_hbm.at[pl.ds(base, _CH)], src_v)
            pltpu.sync_copy(dst_hbm.at[pl.ds(base, _CH)], dst_v)
            pltpu.async_copy(xp_hbm.at[src_v], rows_v, sem).wait()
            pltpu.sync_copy(rows_v, acc_sh.at[dst_v], add=True)
            return carry

        lax.fori_loop(0, _NCHUNK, body, 0)
        plsc.subcore_barrier()
        pltpu.sync_copy(acc_sh.at[pl.ds(sid * _RPT, _RPT), :],
                        out_hbm.at[cid, pl.ds(sid * _RPT, _RPT), :])

    _SC_CACHE["k"] = (sc_counts, sc_spmm)
    return _SC_CACHE["k"]


def _tc_prep_body(cnt_ref, x_ref, dinv_ref, xp_ref):
    deg = cnt_ref[0:1, :] + cnt_ref[1:2, :] + 1.0
    dinv_row = lax.rsqrt(deg)
    eye = (lax.broadcasted_iota(jnp.int32, (128, 128), 0)
           == lax.broadcasted_iota(jnp.int32, (128, 128), 1)).astype(jnp.float32)
    blocks = [
        jnp.sum(eye * dinv_row[:, i * 128:(i + 1) * 128], axis=1,
                keepdims=True)
        for i in range(_NP // 128)
    ]
    dinv = jnp.concatenate(blocks, axis=0)
    dinv_ref[...] = dinv
    xp_ref[...] = x_ref[...] * dinv


_tc_prep = pl.pallas_call(
    _tc_prep_body,
    out_shape=(
        jax.ShapeDtypeStruct((_NP, 1), jnp.float32),
        jax.ShapeDtypeStruct((_NP, _D), jnp.float32),
    ),
)


def _tc_layer_body(p_ref, xp_ref, dinv_ref, w_ref, b_ref, g_ref, bt_ref,
                   o_ref):
    dinv = dinv_ref[...]
    t = (p_ref[0] + p_ref[1] + xp_ref[...]) * dinv
    y = jnp.dot(t, w_ref[...], preferred_element_type=jnp.float32) + b_ref[...]
    yr = y[:_N]
    m = jnp.mean(yr, axis=0, keepdims=True)
    v = jnp.mean((yr - m) * (yr - m), axis=0, keepdims=True)
    h = (y - m) * lax.rsqrt(v + _EPS) * g_ref[...] + bt_ref[...]
    o_ref[...] = jnp.maximum(h, 0.0) * dinv


_tc_layer = pl.pallas_call(
    _tc_layer_body,
    out_shape=jax.ShapeDtypeStruct((_NP, _D), jnp.float32),
)


def _tc_final_body(p_ref, xp_ref, dinv_ref, w_ref, b_ref, x_ref, o_ref):
    t = (p_ref[0] + p_ref[1] + xp_ref[...]) * dinv_ref[...]
    y = jnp.dot(t, w_ref[...], preferred_element_type=jnp.float32)
    o_ref[...] = y + b_ref[...] + x_ref[...]


_tc_final = pl.pallas_call(
    _tc_final_body,
    out_shape=jax.ShapeDtypeStruct((_NP, _D), jnp.float32),
)


def kernel(x, edge_index, W1, b1, g1, bt1, W2, b2, g2, bt2, W3, b3):
    sc_counts, sc_spmm = _sc_kernels()
    src = edge_index[0].astype(jnp.int32)
    dst = edge_index[1].astype(jnp.int32)
    pad = jnp.full((_EPAD - _E,), _N, jnp.int32)
    srcp = jnp.concatenate([src, pad])
    dstp = jnp.concatenate([dst, pad])
    xpd = jnp.pad(x, ((0, _NP - _N), (0, 0)))

    cnt = sc_counts(dstp)
    dinv, xp1 = _tc_prep(cnt, xpd)
    p1 = sc_spmm(xp1, srcp, dstp)
    xp2 = _tc_layer(p1, xp1, dinv, W1, b1.reshape(1, _D),
                    g1.reshape(1, _D), bt1.reshape(1, _D))
    p2 = sc_spmm(xp2, srcp, dstp)
    xp3 = _tc_layer(p2, xp2, dinv, W2, b2.reshape(1, _D),
                    g2.reshape(1, _D), bt2.reshape(1, _D))
    p3 = sc_spmm(xp3, srcp, dstp)
    out = _tc_final(p3, xp3, dinv, W3, b3.reshape(1, _D), xpd)
    return out[:_N]


# speedup vs baseline: 8.3346x; 1.0488x over previous
import functools

import jax
import jax.numpy as jnp
from jax import lax
from jax.experimental import pallas as pl
from jax.experimental.pallas import tpu as pltpu, tpu_sc as plsc

_N = 10000
_D = 128
_E = 320000
_EPS = 1e-5

_NP = 10240
_CH = 128
_NCHUNK = 80
_EPT = _CH * _NCHUNK
_EPAD = 32 * _EPT
_RPT = _NP // 16

_SC_CACHE = {}


def _sc_kernels():
    if "k" in _SC_CACHE:
        return _SC_CACHE["k"]
    mesh = plsc.VectorSubcoreMesh(core_axis_name="c", subcore_axis_name="s")

    @functools.partial(
        pl.kernel,
        mesh=mesh,
        out_type=jax.ShapeDtypeStruct((2, _NP), jnp.float32),
        scratch_types=[
            pltpu.VMEM((_EPT,), jnp.int32),
            pltpu.VMEM((_CH,), jnp.int32),
            pltpu.VMEM((_CH,), jnp.float32),
            pltpu.VMEM((_RPT,), jnp.float32),
            pltpu.VMEM_SHARED((_NP,), jnp.float32),
        ],
    )
    def sc_counts(dst_hbm, out_hbm, dst_1d, dst_c, ones_v, zero_v, acc_sh):
        cid = lax.axis_index("c")
        sid = lax.axis_index("s")
        wid = sid * 2 + cid
        for k in range(_CH // 16):
            ones_v[pl.ds(k * 16, 16)] = jnp.ones((16,), jnp.float32)

        def zbody(k, carry):
            zero_v[pl.ds(k * 16, 16)] = jnp.zeros((16,), jnp.float32)
            return carry

        lax.fori_loop(0, _RPT // 16, zbody, 0)
        pltpu.sync_copy(zero_v, acc_sh.at[pl.ds(sid * _RPT, _RPT)])
        pltpu.sync_copy(dst_hbm.at[pl.ds(wid * _EPT, _EPT)], dst_1d)
        plsc.subcore_barrier()

        def body(j, carry):
            for k in range(_CH // 16):
                dst_c[pl.ds(k * 16, 16)] = dst_1d[pl.ds(j * _CH + k * 16, 16)]
            pltpu.sync_copy(ones_v, acc_sh.at[dst_c], add=True)
            return carry

        lax.fori_loop(0, _NCHUNK, body, 0)
        plsc.subcore_barrier()
        pltpu.sync_copy(acc_sh.at[pl.ds(sid * _RPT, _RPT)],
                        out_hbm.at[cid, pl.ds(sid * _RPT, _RPT)])

    @functools.partial(
        pl.kernel,
        mesh=mesh,
        out_type=jax.ShapeDtypeStruct((2, _NP, _D), jnp.float32),
        scratch_types=[
            pltpu.VMEM((_CH,), jnp.int32),
            pltpu.VMEM((_CH,), jnp.int32),
            pltpu.VMEM((_CH,), jnp.int32),
            pltpu.VMEM((_CH,), jnp.int32),
            pltpu.VMEM((_CH, _D), jnp.float32),
            pltpu.VMEM((_CH, _D), jnp.float32),
            pltpu.VMEM_SHARED((_NP, _D), jnp.float32),
            pltpu.SemaphoreType.DMA,
            pltpu.SemaphoreType.DMA,
        ],
    )
    def sc_spmm(xp_hbm, src_hbm, dst_hbm, out_hbm,
                src_a, src_b, dst_a, dst_b, rows_a, rows_b, acc_sh,
                sem_a, sem_b):
        cid = lax.axis_index("c")
        sid = lax.axis_index("s")
        wid = sid * 2 + cid

        def zbody(r, carry):
            for k in range(_D // 16):
                rows_a[r, pl.ds(k * 16, 16)] = jnp.zeros((16,), jnp.float32)
            return carry

        lax.fori_loop(0, _CH, zbody, 0)
        for k in range(_RPT // _CH):
            pltpu.sync_copy(rows_a,
                            acc_sh.at[pl.ds(sid * _RPT + k * _CH, _CH), :])
        plsc.subcore_barrier()

        def fetch(j, src_v, dst_v, buf, sem):
            base = wid * _EPT + j * _CH
            pltpu.sync_copy(src_hbm.at[pl.ds(base, _CH)], src_v)
            pltpu.sync_copy(dst_hbm.at[pl.ds(base, _CH)], dst_v)
            pltpu.async_copy(xp_hbm.at[src_v], buf, sem)

        def gwait(src_v, buf, sem):
            pltpu.make_async_copy(xp_hbm.at[src_v], buf, sem).wait()

        fetch(0, src_a, dst_a, rows_a, sem_a)

        def body(m, carry):
            ja = 2 * m
            jb = 2 * m + 1
            fetch(jb, src_b, dst_b, rows_b, sem_b)
            gwait(src_a, rows_a, sem_a)
            pltpu.sync_copy(rows_a, acc_sh.at[dst_a], add=True)

            @pl.when(m < _NCHUNK // 2 - 1)
            def _():
                fetch(ja + 2, src_a, dst_a, rows_a, sem_a)

            gwait(src_b, rows_b, sem_b)
            pltpu.sync_copy(rows_b, acc_sh.at[dst_b], add=True)
            return carry

        lax.fori_loop(0, _NCHUNK // 2, body, 0)
        plsc.subcore_barrier()
        pltpu.sync_copy(acc_sh.at[pl.ds(sid * _RPT, _RPT), :],
                        out_hbm.at[cid, pl.ds(sid * _RPT, _RPT), :])

    _SC_CACHE["k"] = (sc_counts, sc_spmm)
    return _SC_CACHE["k"]


def _tc_prep_body(cnt_ref, x_ref, dinv_ref, xp_ref):
    deg = cnt_ref[0:1, :] + cnt_ref[1:2, :] + 1.0
    dinv_row = lax.rsqrt(deg)
    eye = (lax.broadcasted_iota(jnp.int32, (128, 128), 0)
           == lax.broadcasted_iota(jnp.int32, (128, 128), 1)).astype(jnp.float32)
    blocks = [
        jnp.sum(eye * dinv_row[:, i * 128:(i + 1) * 128], axis=1,
                keepdims=True)
        for i in range(_NP // 128)
    ]
    dinv = jnp.concatenate(blocks, axis=0)
    dinv_ref[...] = dinv
    xp_ref[...] = x_ref[...] * dinv


_tc_prep = pl.pallas_call(
    _tc_prep_body,
    out_shape=(
        jax.ShapeDtypeStruct((_NP, 1), jnp.float32),
        jax.ShapeDtypeStruct((_NP, _D), jnp.float32),
    ),
)


def _tc_layer_body(p_ref, xp_ref, dinv_ref, w_ref, b_ref, g_ref, bt_ref,
                   o_ref):
    dinv = dinv_ref[...]
    t = (p_ref[0] + p_ref[1] + xp_ref[...]) * dinv
    y = jnp.dot(t, w_ref[...], preferred_element_type=jnp.float32) + b_ref[...]
    yr = y[:_N]
    m = jnp.mean(yr, axis=0, keepdims=True)
    v = jnp.mean((yr - m) * (yr - m), axis=0, keepdims=True)
    h = (y - m) * lax.rsqrt(v + _EPS) * g_ref[...] + bt_ref[...]
    o_ref[...] = jnp.maximum(h, 0.0) * dinv


_tc_layer = pl.pallas_call(
    _tc_layer_body,
    out_shape=jax.ShapeDtypeStruct((_NP, _D), jnp.float32),
)


def _tc_final_body(p_ref, xp_ref, dinv_ref, w_ref, b_ref, x_ref, o_ref):
    t = (p_ref[0] + p_ref[1] + xp_ref[...]) * dinv_ref[...]
    y = jnp.dot(t, w_ref[...], preferred_element_type=jnp.float32)
    o_ref[...] = y + b_ref[...] + x_ref[...]


_tc_final = pl.pallas_call(
    _tc_final_body,
    out_shape=jax.ShapeDtypeStruct((_NP, _D), jnp.float32),
)


def kernel(x, edge_index, W1, b1, g1, bt1, W2, b2, g2, bt2, W3, b3):
    sc_counts, sc_spmm = _sc_kernels()
    src = edge_index[0].astype(jnp.int32)
    dst = edge_index[1].astype(jnp.int32)
    pad = jnp.full((_EPAD - _E,), _N, jnp.int32)
    srcp = jnp.concatenate([src, pad])
    dstp = jnp.concatenate([dst, pad])
    xpd = jnp.pad(x, ((0, _NP - _N), (0, 0)))

    cnt = sc_counts(dstp)
    dinv, xp1 = _tc_prep(cnt, xpd)
    p1 = sc_spmm(xp1, srcp, dstp)
    xp2 = _tc_layer(p1, xp1, dinv, W1, b1.reshape(1, _D),
                    g1.reshape(1, _D), bt1.reshape(1, _D))
    p2 = sc_spmm(xp2, srcp, dstp)
    xp3 = _tc_layer(p2, xp2, dinv, W2, b2.reshape(1, _D),
                    g2.reshape(1, _D), bt2.reshape(1, _D))
    p3 = sc_spmm(xp3, srcp, dstp)
    out = _tc_final(p3, xp3, dinv, W3, b3.reshape(1, _D), xpd)
    return out[:_N]
